# Initial kernel scaffold; baseline (speedup 1.0000x reference)
import jax
import jax.numpy as jnp
from jax.experimental import pallas as pl


def kernel(boxes, classification, centerness):
    raise NotImplementedError("write your pallas kernel here")


# speedup vs baseline: 269.7322x; 269.7322x over previous
import functools

import jax
import jax.numpy as jnp
from jax import lax
from jax.experimental import pallas as pl
from jax.experimental.pallas import tpu as pltpu
from jax.experimental.pallas import tpu_sc as plsc

_B, _N, _C = 8, 5000, 80
_SCORE_T = 0.05
_NMS_T = 0.5
_MAXDET = 300
_NP = 5120
_N16 = 5008
_NS = 5024
_KCAP = 304
_TI, _TJ = 512, 1024

_NEG_INF = float("-inf")


def _prep_body(cls_ref, cen_ref, rank_ref, pos_ref, lab_ref, cnt_scr, kp_scr):
    cls = cls_ref[...]
    raw = jnp.max(cls, axis=1)
    iota_c = lax.broadcasted_iota(jnp.int32, (_N, _C), 1)
    lab = jnp.min(jnp.where(cls == raw[:, None], iota_c, _C), axis=1)
    cen = cen_ref[0, :]
    comb = jnp.sqrt(raw * cen)
    valid = raw > _SCORE_T
    rank = jnp.where(valid, comb, _NEG_INF)
    rank_ref[0, :] = rank
    lab_ref[0, :] = lab

    k = lax.bitcast_convert_type(rank, jnp.int32)
    kneg = lax.bitcast_convert_type(
        jnp.float32(_NEG_INF), jnp.int32)
    kp_scr[pl.ds(0, _N)] = k
    kp_scr[pl.ds(_N, _NP - _N)] = jnp.full((_NP - _N,), kneg, jnp.int32)

    c0 = (lax.broadcasted_iota(jnp.int32, (_TI, _TJ), 1)
          - lax.broadcasted_iota(jnp.int32, (_TI, _TJ), 0))

    def it_body(it, carry):
        a = kp_scr[pl.ds(it * _TI, _TI)].reshape(_TI, 1)

        def jt_body(jt, acc):
            bj = kp_scr[pl.ds(jt * _TJ, _TJ)].reshape(1, _TJ)
            diff = bj - a
            d = it * _TI - jt * _TJ
            t = jnp.where(c0 < d, 0, 1)
            return acc + jnp.where(diff >= t, 1, 0)

        acc = lax.fori_loop(0, _NP // _TJ, jt_body,
                            jnp.zeros((_TI, _TJ), jnp.int32))
        cnt_scr[pl.ds(it * _TI, _TI)] = jnp.sum(acc, axis=1)
        return carry

    lax.fori_loop(0, _NP // _TI, it_body, 0)
    pos_ref[0, :] = cnt_scr[pl.ds(0, _N)]


def _prep(classification, centerness):
    rank, pos, lab = pl.pallas_call(
        _prep_body,
        grid=(_B,),
        in_specs=[
            pl.BlockSpec((None, _N, _C), lambda b: (b, 0, 0)),
            pl.BlockSpec((None, 1, _N), lambda b: (b, 0, 0)),
        ],
        out_specs=[
            pl.BlockSpec((None, 1, _N), lambda b: (b, 0, 0)),
            pl.BlockSpec((None, 1, _N), lambda b: (b, 0, 0)),
            pl.BlockSpec((None, 1, _N), lambda b: (b, 0, 0)),
        ],
        out_shape=[
            jax.ShapeDtypeStruct((_B, 1, _N), jnp.float32),
            jax.ShapeDtypeStruct((_B, 1, _N), jnp.int32),
            jax.ShapeDtypeStruct((_B, 1, _N), jnp.int32),
        ],
        scratch_shapes=[pltpu.VMEM((_NP,), jnp.int32),
                        pltpu.VMEM((_NP,), jnp.int32)],
    )(classification, centerness[:, None, :])
    return (rank.reshape(_B, _N), pos.reshape(_B, _N), lab.reshape(_B, _N))


def _sc_body(rank_h, pos_h, lab_h, boxt_h,
             ob_h, os_h, ol_h,
             in_rank, in_pos, in_lab, in_box,
             sr, slb, sx1, sy1, sx2, sy2, sar,
             kx1, ky1, kx2, ky2, kar,
             obox, oscr, olab):
    bb = lax.axis_index("c") + 2 * lax.axis_index("s")

    @pl.when(bb < _B)
    def _work():
        pltpu.sync_copy(rank_h.at[bb], in_rank.at[pl.ds(0, _N)])
        pltpu.sync_copy(pos_h.at[bb], in_pos.at[pl.ds(0, _N)])
        pltpu.sync_copy(lab_h.at[bb], in_lab.at[pl.ds(0, _N)])
        pltpu.sync_copy(boxt_h.at[bb], in_box)

        lanes = lax.iota(jnp.int32, 16)
        lane0 = lanes == 0
        neg1f = jnp.full((16,), -1.0, jnp.float32)
        neg1i = jnp.full((16,), -1, jnp.int32)
        ninf = jnp.full((16,), _NEG_INF, jnp.float32)

        sr[pl.ds(4992, 16)] = ninf
        sr[pl.ds(5008, 16)] = ninf

        def init_k(m, carry):
            ms = pl.ds(m * 16, 16)
            kx1[ms] = neg1f
            ky1[ms] = neg1f
            kx2[ms] = neg1f
            ky2[ms] = neg1f
            kar[ms] = jnp.zeros((16,), jnp.float32)
            return carry

        lax.fori_loop(0, _KCAP // 16, init_k, 0)

        def init_o(m, carry):
            ms = pl.ds(m * 16, 16)
            oscr[ms] = neg1f
            olab[ms] = neg1i
            return carry

        lax.fori_loop(0, _KCAP // 16, init_o, 0)

        def init_ob(m, carry):
            obox[pl.ds(m * 16, 16)] = neg1f
            return carry

        lax.fori_loop(0, (_KCAP * 4) // 16, init_ob, 0)

        def scatter_chunk(k, mask):
            ms = pl.ds(k * 16, 16)
            idx = in_pos[ms]
            plsc.store_scatter(sr, [idx], in_rank[ms], mask=mask)
            plsc.store_scatter(slb, [idx], in_lab[ms], mask=mask)
            x1 = in_box[0, ms]
            y1 = in_box[1, ms]
            x2 = in_box[2, ms]
            y2 = in_box[3, ms]
            plsc.store_scatter(sx1, [idx], x1, mask=mask)
            plsc.store_scatter(sy1, [idx], y1, mask=mask)
            plsc.store_scatter(sx2, [idx], x2, mask=mask)
            plsc.store_scatter(sy2, [idx], y2, mask=mask)
            plsc.store_scatter(sar, [idx], (x2 - x1) * (y2 - y1), mask=mask)

        full = lanes < 16

        def sc_body(k, carry):
            scatter_chunk(k, full)
            return carry

        lax.fori_loop(0, _N // 16, sc_body, 0)
        scatter_chunk(_N // 16, lanes < (_N % 16))

        def nms_cond(st):
            return st[2]

        def nms_body(st):
            ci, nk, _go = st
            cs = pl.ds(ci, 16)
            csc = sr[cs][0]
            clb = slb[cs][0]
            cx1 = sx1[cs][0]
            cy1 = sy1[cs][0]
            cx2 = sx2[cs][0]
            cy2 = sy2[cs][0]
            car = sar[cs][0]

            def ch_body(m, sup):
                ms = pl.ds(m * 16, 16)
                ix1 = jnp.maximum(kx1[ms], cx1)
                iy1 = jnp.maximum(ky1[ms], cy1)
                ix2 = jnp.minimum(kx2[ms], cx2)
                iy2 = jnp.minimum(ky2[ms], cy2)
                inter = (jnp.maximum(ix2 - ix1, 0.0)
                         * jnp.maximum(iy2 - iy1, 0.0))
                union = kar[ms] + car - inter
                iou = inter / union
                return sup | jnp.any(iou > _NMS_T)

            nchunks = (nk + 15) // 16
            sup = lax.fori_loop(0, nchunks, ch_body, False)

            @pl.when(jnp.logical_not(sup))
            def _append():
                nkv = jnp.full((16,), nk, jnp.int32)
                plsc.store_scatter(kx1, [nkv], jnp.full((16,), cx1), mask=lane0)
                plsc.store_scatter(ky1, [nkv], jnp.full((16,), cy1), mask=lane0)
                plsc.store_scatter(kx2, [nkv], jnp.full((16,), cx2), mask=lane0)
                plsc.store_scatter(ky2, [nkv], jnp.full((16,), cy2), mask=lane0)
                plsc.store_scatter(kar, [nkv], jnp.full((16,), car), mask=lane0)
                plsc.store_scatter(oscr, [nkv], jnp.full((16,), csc), mask=lane0)
                plsc.store_scatter(olab, [nkv], jnp.full((16,), clb), mask=lane0)
                bv = jnp.where(lanes == 0, cx1,
                               jnp.where(lanes == 1, cy1,
                                         jnp.where(lanes == 2, cx2, cy2)))
                plsc.store_scatter(obox, [nk * 4 + lanes], bv, mask=lanes < 4)

            nk2 = nk + jnp.where(sup, 0, 1)
            ci2 = ci + 1
            go2 = (ci2 < _N) & (nk2 < _MAXDET) & (sr[pl.ds(ci2, 16)][0] > _NEG_INF)
            return ci2, nk2, go2

        go0 = sr[pl.ds(0, 16)][0] > _NEG_INF
        lax.while_loop(nms_cond, nms_body,
                       (jnp.int32(0), jnp.int32(0), go0))

        pltpu.sync_copy(obox, ob_h.at[bb])
        pltpu.sync_copy(oscr, os_h.at[bb])
        pltpu.sync_copy(olab, ol_h.at[bb])


def _sc_nms(rank, pos, lab, boxt):
    mesh = plsc.VectorSubcoreMesh(core_axis_name="c", subcore_axis_name="s")
    f = pl.kernel(
        _sc_body,
        out_type=[
            jax.ShapeDtypeStruct((_B, _KCAP * 4), jnp.float32),
            jax.ShapeDtypeStruct((_B, _KCAP), jnp.float32),
            jax.ShapeDtypeStruct((_B, _KCAP), jnp.int32),
        ],
        mesh=mesh,
        compiler_params=pltpu.CompilerParams(needs_layout_passes=False,
                                             use_tc_tiling_on_sc=False),
        scratch_types=[
            pltpu.VMEM((_N16,), jnp.float32),
            pltpu.VMEM((_N16,), jnp.int32),
            pltpu.VMEM((_N16,), jnp.int32),
            pltpu.VMEM((4, _N16), jnp.float32),
            pltpu.VMEM((_NS,), jnp.float32),
            pltpu.VMEM((_NS,), jnp.int32),
            pltpu.VMEM((_NS,), jnp.float32),
            pltpu.VMEM((_NS,), jnp.float32),
            pltpu.VMEM((_NS,), jnp.float32),
            pltpu.VMEM((_NS,), jnp.float32),
            pltpu.VMEM((_NS,), jnp.float32),
            pltpu.VMEM((_KCAP,), jnp.float32),
            pltpu.VMEM((_KCAP,), jnp.float32),
            pltpu.VMEM((_KCAP,), jnp.float32),
            pltpu.VMEM((_KCAP,), jnp.float32),
            pltpu.VMEM((_KCAP,), jnp.float32),
            pltpu.VMEM((_KCAP * 4,), jnp.float32),
            pltpu.VMEM((_KCAP,), jnp.float32),
            pltpu.VMEM((_KCAP,), jnp.int32),
        ],
    )
    return f(rank, pos, lab, boxt)


def kernel(boxes, classification, centerness):
    boxt = jnp.transpose(boxes, (0, 2, 1))
    boxt = jnp.concatenate(
        [boxt, jnp.zeros((_B, 4, _N16 - _N), jnp.float32)], axis=2)
    cen = centerness[..., 0]
    rank, pos, lab = _prep(classification, cen)
    ob, osr, olb = _sc_nms(rank, pos, lab, boxt)
    boxes_out = ob.reshape(_B, _KCAP, 4)[:, :_MAXDET, :]
    scores_out = osr[:, :_MAXDET]
    labels_out = olb[:, :_MAXDET]
    return boxes_out, scores_out, labels_out


# speedup vs baseline: 906.5184x; 3.3608x over previous
import functools

import jax
import jax.numpy as jnp
from jax import lax
from jax.experimental import pallas as pl
from jax.experimental.pallas import tpu as pltpu
from jax.experimental.pallas import tpu_sc as plsc

_B, _N, _C = 8, 5000, 80
_SCORE_T = 0.05
_NMS_T = 0.5
_MAXDET = 300
_NSORT = 8192
_SR, _SC = 64, 128
_N16 = 5008
_NS = 5024
_KCAP = 304

_NEG_INF = float("-inf")


def _partner(x, j, axis):
    lo = jnp.roll(x, -j, axis=axis)
    hi = jnp.roll(x, j, axis=axis)
    idx = lax.broadcasted_iota(jnp.int32, (_SR, _SC), axis)
    return jnp.where((idx & j) == 0, lo, hi), (idx & j) == 0


def _prep_body(cls_ref, cen_ref, skey_ref, sidx_ref, lab_ref):
    cls = cls_ref[...]
    raw = jnp.max(cls, axis=1)
    iota_c = lax.broadcasted_iota(jnp.int32, (_N, _C), 1)
    lab = jnp.min(jnp.where(cls == raw[:, None], iota_c, _C), axis=1)
    cen = cen_ref[0, :]
    comb = jnp.sqrt(raw * cen)
    valid = raw > _SCORE_T
    rank = jnp.where(valid, comb, _NEG_INF)
    lab_ref[0, :] = lab

    k = lax.bitcast_convert_type(rank, jnp.int32)
    kneg = lax.bitcast_convert_type(jnp.float32(_NEG_INF), jnp.int32)
    K = jnp.concatenate(
        [k, jnp.full((_NSORT - _N,), kneg, jnp.int32)]).reshape(_SR, _SC)
    I = (lax.broadcasted_iota(jnp.int32, (_SR, _SC), 0) * _SC
         + lax.broadcasted_iota(jnp.int32, (_SR, _SC), 1))

    row_i = lax.broadcasted_iota(jnp.int32, (_SR, _SC), 0)
    lane_i = lax.broadcasted_iota(jnp.int32, (_SR, _SC), 1)

    kk = 2
    while kk <= _NSORT:
        if kk >= _SC:
            up = (row_i & (kk // _SC)) == 0
        else:
            up = (lane_i & kk) == 0
        jj = kk // 2
        while jj >= 1:
            if jj >= _SC:
                pK, lower = _partner(K, jj // _SC, 0)
                pI, _ = _partner(I, jj // _SC, 0)
            else:
                pK, lower = _partner(K, jj, 1)
                pI, _ = _partner(I, jj, 1)
            a_lt_b = (K > pK) | ((K == pK) & (I < pI))
            keep_small = lower == up
            take_self = a_lt_b == keep_small
            K = jnp.where(take_self, K, pK)
            I = jnp.where(take_self, I, pI)
            jj //= 2
        kk *= 2

    skey_ref[0, :] = lax.bitcast_convert_type(
        K.reshape(_NSORT), jnp.float32)
    sidx_ref[0, :] = I.reshape(_NSORT)


def _prep(classification, centerness):
    skey, sidx, lab = pl.pallas_call(
        _prep_body,
        grid=(_B,),
        in_specs=[
            pl.BlockSpec((None, _N, _C), lambda b: (b, 0, 0)),
            pl.BlockSpec((None, 1, _N), lambda b: (b, 0, 0)),
        ],
        out_specs=[
            pl.BlockSpec((None, 1, _NSORT), lambda b: (b, 0, 0)),
            pl.BlockSpec((None, 1, _NSORT), lambda b: (b, 0, 0)),
            pl.BlockSpec((None, 1, _N), lambda b: (b, 0, 0)),
        ],
        out_shape=[
            jax.ShapeDtypeStruct((_B, 1, _NSORT), jnp.float32),
            jax.ShapeDtypeStruct((_B, 1, _NSORT), jnp.int32),
            jax.ShapeDtypeStruct((_B, 1, _N), jnp.int32),
        ],
    )(classification, centerness[:, None, :])
    return (skey.reshape(_B, _NSORT), sidx.reshape(_B, _NSORT),
            lab.reshape(_B, _N))


def _sc_body(skey_h, sidx_h, lab_h, boxt_h,
             ob_h, os_h, ol_h,
             in_sidx, in_lab, in_box,
             sr, slb, sx1, sy1, sx2, sy2, sar,
             kx1, ky1, kx2, ky2, kar,
             obox, oscr, olab):
    bb = lax.axis_index("c") + 2 * lax.axis_index("s")

    @pl.when(bb < _B)
    def _work():
        pltpu.sync_copy(skey_h.at[bb, pl.ds(0, _NS)], sr)
        pltpu.sync_copy(sidx_h.at[bb, pl.ds(0, _N16)], in_sidx)
        pltpu.sync_copy(lab_h.at[bb], in_lab.at[pl.ds(0, _N)])
        pltpu.sync_copy(boxt_h.at[bb], in_box)

        lanes = lax.iota(jnp.int32, 16)
        lane0 = lanes == 0
        neg1f = jnp.full((16,), -1.0, jnp.float32)
        neg1i = jnp.full((16,), -1, jnp.int32)

        def init_k(m, carry):
            ms = pl.ds(m * 16, 16)
            kx1[ms] = neg1f
            ky1[ms] = neg1f
            kx2[ms] = neg1f
            ky2[ms] = neg1f
            kar[ms] = jnp.zeros((16,), jnp.float32)
            return carry

        lax.fori_loop(0, _KCAP // 16, init_k, 0)

        def init_o(m, carry):
            ms = pl.ds(m * 16, 16)
            oscr[ms] = neg1f
            olab[ms] = neg1i
            return carry

        lax.fori_loop(0, _KCAP // 16, init_o, 0)

        def init_ob(m, carry):
            obox[pl.ds(m * 16, 16)] = neg1f
            return carry

        lax.fori_loop(0, (_KCAP * 4) // 16, init_ob, 0)

        row0 = jnp.zeros((16,), jnp.int32)

        def gather_chunk(k, carry):
            ms = pl.ds(k * 16, 16)
            idx = in_sidx[ms]
            slb[ms] = plsc.load_gather(in_lab, [idx])
            x1 = plsc.load_gather(in_box, [row0, idx])
            y1 = plsc.load_gather(in_box, [row0 + 1, idx])
            x2 = plsc.load_gather(in_box, [row0 + 2, idx])
            y2 = plsc.load_gather(in_box, [row0 + 3, idx])
            sx1[ms] = x1
            sy1[ms] = y1
            sx2[ms] = x2
            sy2[ms] = y2
            sar[ms] = (x2 - x1) * (y2 - y1)
            return carry

        lax.fori_loop(0, _N16 // 16, gather_chunk, 0)

        def nms_cond(st):
            return st[2]

        def nms_body(st):
            ci, nk, _go = st
            cs = pl.ds(ci, 16)
            csc = sr[cs][0]
            clb = slb[cs][0]
            cx1 = sx1[cs][0]
            cy1 = sy1[cs][0]
            cx2 = sx2[cs][0]
            cy2 = sy2[cs][0]
            car = sar[cs][0]

            def ch_body(m, sup):
                ms = pl.ds(m * 16, 16)
                ix1 = jnp.maximum(kx1[ms], cx1)
                iy1 = jnp.maximum(ky1[ms], cy1)
                ix2 = jnp.minimum(kx2[ms], cx2)
                iy2 = jnp.minimum(ky2[ms], cy2)
                inter = (jnp.maximum(ix2 - ix1, 0.0)
                         * jnp.maximum(iy2 - iy1, 0.0))
                union = kar[ms] + car - inter
                iou = inter / union
                return sup | jnp.any(iou > _NMS_T)

            nchunks = (nk + 15) // 16
            sup = lax.fori_loop(0, nchunks, ch_body, False)

            @pl.when(jnp.logical_not(sup))
            def _append():
                nkv = jnp.full((16,), nk, jnp.int32)
                plsc.store_scatter(kx1, [nkv], jnp.full((16,), cx1), mask=lane0)
                plsc.store_scatter(ky1, [nkv], jnp.full((16,), cy1), mask=lane0)
                plsc.store_scatter(kx2, [nkv], jnp.full((16,), cx2), mask=lane0)
                plsc.store_scatter(ky2, [nkv], jnp.full((16,), cy2), mask=lane0)
                plsc.store_scatter(kar, [nkv], jnp.full((16,), car), mask=lane0)
                plsc.store_scatter(oscr, [nkv], jnp.full((16,), csc), mask=lane0)
                plsc.store_scatter(olab, [nkv], jnp.full((16,), clb), mask=lane0)
                bv = jnp.where(lanes == 0, cx1,
                               jnp.where(lanes == 1, cy1,
                                         jnp.where(lanes == 2, cx2, cy2)))
                plsc.store_scatter(obox, [nk * 4 + lanes], bv, mask=lanes < 4)

            nk2 = nk + jnp.where(sup, 0, 1)
            ci2 = ci + 1
            go2 = (ci2 < _N) & (nk2 < _MAXDET) & (sr[pl.ds(ci2, 16)][0] > _NEG_INF)
            return ci2, nk2, go2

        go0 = sr[pl.ds(0, 16)][0] > _NEG_INF
        lax.while_loop(nms_cond, nms_body,
                       (jnp.int32(0), jnp.int32(0), go0))

        pltpu.sync_copy(obox, ob_h.at[bb])
        pltpu.sync_copy(oscr, os_h.at[bb])
        pltpu.sync_copy(olab, ol_h.at[bb])


def _sc_nms(skey, sidx, lab, boxt):
    mesh = plsc.VectorSubcoreMesh(core_axis_name="c", subcore_axis_name="s")
    f = pl.kernel(
        _sc_body,
        out_type=[
            jax.ShapeDtypeStruct((_B, _KCAP * 4), jnp.float32),
            jax.ShapeDtypeStruct((_B, _KCAP), jnp.float32),
            jax.ShapeDtypeStruct((_B, _KCAP), jnp.int32),
        ],
        mesh=mesh,
        compiler_params=pltpu.CompilerParams(needs_layout_passes=False,
                                             use_tc_tiling_on_sc=False),
        scratch_types=[
            pltpu.VMEM((_N16,), jnp.int32),
            pltpu.VMEM((_N16,), jnp.int32),
            pltpu.VMEM((4, _N16), jnp.float32),
            pltpu.VMEM((_NS,), jnp.float32),
            pltpu.VMEM((_NS,), jnp.int32),
            pltpu.VMEM((_NS,), jnp.float32),
            pltpu.VMEM((_NS,), jnp.float32),
            pltpu.VMEM((_NS,), jnp.float32),
            pltpu.VMEM((_NS,), jnp.float32),
            pltpu.VMEM((_NS,), jnp.float32),
            pltpu.VMEM((_KCAP,), jnp.float32),
            pltpu.VMEM((_KCAP,), jnp.float32),
            pltpu.VMEM((_KCAP,), jnp.float32),
            pltpu.VMEM((_KCAP,), jnp.float32),
            pltpu.VMEM((_KCAP,), jnp.float32),
            pltpu.VMEM((_KCAP * 4,), jnp.float32),
            pltpu.VMEM((_KCAP,), jnp.float32),
            pltpu.VMEM((_KCAP,), jnp.int32),
        ],
    )
    return f(skey, sidx, lab, boxt)


def kernel(boxes, classification, centerness):
    boxt = jnp.transpose(boxes, (0, 2, 1))
    boxt = jnp.concatenate(
        [boxt, jnp.zeros((_B, 4, _N16 - _N), jnp.float32)], axis=2)
    cen = centerness[..., 0]
    skey, sidx, lab = _prep(classification, cen)
    ob, osr, olb = _sc_nms(skey, sidx, lab, boxt)
    boxes_out = ob.reshape(_B, _KCAP, 4)[:, :_MAXDET, :]
    scores_out = osr[:, :_MAXDET]
    labels_out = olb[:, :_MAXDET]
    return boxes_out, scores_out, labels_out


# speedup vs baseline: 1015.3636x; 1.1201x over previous
import functools

import jax
import jax.numpy as jnp
from jax import lax
from jax.experimental import pallas as pl
from jax.experimental.pallas import tpu as pltpu
from jax.experimental.pallas import tpu_sc as plsc

_B, _N, _C = 8, 5000, 80
_SCORE_T = 0.05
_NMS_T = 0.5
_MAXDET = 300
_NSORT = 8192
_BR = 64
_SR, _SC = 512, 128
_N16 = 5008
_NS = 5024
_KCAP = 304

_NEG_INF = float("-inf")


def _partner(x, j, axis):
    lo = jnp.roll(x, -j, axis=axis)
    hi = jnp.roll(x, j, axis=axis)
    idx = lax.broadcasted_iota(jnp.int32, (_SR, _SC), axis)
    return jnp.where((idx & j) == 0, lo, hi), (idx & j) == 0


def _reduce_body(cls_ref, cen_ref, rank_ref, lab_ref):
    cls = cls_ref[...]
    raw = jnp.max(cls, axis=1)
    iota_c = lax.broadcasted_iota(jnp.int32, (_N, _C), 1)
    lab = jnp.min(jnp.where(cls == raw[:, None], iota_c, _C), axis=1)
    cen = cen_ref[0, :]
    comb = jnp.sqrt(raw * cen)
    valid = raw > _SCORE_T
    rank_ref[0, :] = jnp.where(valid, comb, _NEG_INF)
    lab_ref[0, :] = lab


def _sort_body(rank_ref, skey_ref, sidx_ref):
    kneg = lax.bitcast_convert_type(jnp.float32(_NEG_INF), jnp.int32)
    krows = []
    for b in range(_B):
        k = lax.bitcast_convert_type(rank_ref[b], jnp.int32)
        krows.append(jnp.concatenate(
            [k, jnp.full((_NSORT - _N,), kneg, jnp.int32)]).reshape(_BR, _SC))

    K = jnp.concatenate(krows, axis=0)
    row_i = lax.broadcasted_iota(jnp.int32, (_SR, _SC), 0)
    lane_i = lax.broadcasted_iota(jnp.int32, (_SR, _SC), 1)
    brow_i = row_i & (_BR - 1)
    I = brow_i * _SC + lane_i

    kk = 2
    while kk <= _NSORT:
        if kk >= _SC:
            up = (brow_i & (kk // _SC)) == 0
        else:
            up = (lane_i & kk) == 0
        jj = kk // 2
        while jj >= 1:
            if jj >= _SC:
                pK, lower = _partner(K, jj // _SC, 0)
                pI, _ = _partner(I, jj // _SC, 0)
            else:
                pK, lower = _partner(K, jj, 1)
                pI, _ = _partner(I, jj, 1)
            a_lt_b = (K > pK) | ((K == pK) & (I < pI))
            keep_small = lower == up
            take_self = a_lt_b == keep_small
            K = jnp.where(take_self, K, pK)
            I = jnp.where(take_self, I, pI)
            jj //= 2
        kk *= 2

    skey_ref[...] = lax.bitcast_convert_type(K, jnp.float32)
    sidx_ref[...] = I


def _prep(classification, centerness):
    rank, lab = pl.pallas_call(
        _reduce_body,
        grid=(_B,),
        in_specs=[
            pl.BlockSpec((None, _N, _C), lambda b: (b, 0, 0)),
            pl.BlockSpec((None, 1, _N), lambda b: (b, 0, 0)),
        ],
        out_specs=[
            pl.BlockSpec((None, 1, _N), lambda b: (b, 0, 0)),
            pl.BlockSpec((None, 1, _N), lambda b: (b, 0, 0)),
        ],
        out_shape=[
            jax.ShapeDtypeStruct((_B, 1, _N), jnp.float32),
            jax.ShapeDtypeStruct((_B, 1, _N), jnp.int32),
        ],
    )(classification, centerness[:, None, :])
    rank = rank.reshape(_B, _N)
    skey, sidx = pl.pallas_call(
        _sort_body,
        out_shape=[
            jax.ShapeDtypeStruct((_SR, _SC), jnp.float32),
            jax.ShapeDtypeStruct((_SR, _SC), jnp.int32),
        ],
    )(rank)
    return (skey.reshape(_B, _NSORT), sidx.reshape(_B, _NSORT),
            lab.reshape(_B, _N))


def _sc_body(skey_h, sidx_h, lab_h, boxt_h,
             ob_h, os_h, ol_h,
             in_sidx, in_lab, in_box,
             sr, slb, sx1, sy1, sx2, sy2, sar,
             kx1, ky1, kx2, ky2, kar,
             obox, oscr, olab):
    bb = lax.axis_index("c") + 2 * lax.axis_index("s")

    @pl.when(bb < _B)
    def _work():
        pltpu.sync_copy(skey_h.at[bb, pl.ds(0, _NS)], sr)
        pltpu.sync_copy(sidx_h.at[bb, pl.ds(0, _N16)], in_sidx)
        pltpu.sync_copy(lab_h.at[bb], in_lab.at[pl.ds(0, _N)])
        pltpu.sync_copy(boxt_h.at[bb], in_box)

        lanes = lax.iota(jnp.int32, 16)
        lane0 = lanes == 0
        neg1f = jnp.full((16,), -1.0, jnp.float32)
        neg1i = jnp.full((16,), -1, jnp.int32)

        def init_k(m, carry):
            ms = pl.ds(m * 16, 16)
            kx1[ms] = neg1f
            ky1[ms] = neg1f
            kx2[ms] = neg1f
            ky2[ms] = neg1f
            kar[ms] = jnp.zeros((16,), jnp.float32)
            return carry

        lax.fori_loop(0, _KCAP // 16, init_k, 0)

        def init_o(m, carry):
            ms = pl.ds(m * 16, 16)
            oscr[ms] = neg1f
            olab[ms] = neg1i
            return carry

        lax.fori_loop(0, _KCAP // 16, init_o, 0)

        def init_ob(m, carry):
            obox[pl.ds(m * 16, 16)] = neg1f
            return carry

        lax.fori_loop(0, (_KCAP * 4) // 16, init_ob, 0)

        row0 = jnp.zeros((16,), jnp.int32)

        def gather_chunk(k, carry):
            ms = pl.ds(k * 16, 16)
            idx = in_sidx[ms]
            slb[ms] = plsc.load_gather(in_lab, [idx])
            x1 = plsc.load_gather(in_box, [row0, idx])
            y1 = plsc.load_gather(in_box, [row0 + 1, idx])
            x2 = plsc.load_gather(in_box, [row0 + 2, idx])
            y2 = plsc.load_gather(in_box, [row0 + 3, idx])
            sx1[ms] = x1
            sy1[ms] = y1
            sx2[ms] = x2
            sy2[ms] = y2
            sar[ms] = (x2 - x1) * (y2 - y1)
            return carry

        lax.fori_loop(0, _N16 // 16, gather_chunk, 0)

        def nms_cond(st):
            return st[2]

        def nms_body(st):
            ci, nk, _go = st
            cs = pl.ds(ci, 16)
            csc = sr[cs][0]
            clb = slb[cs][0]
            cx1 = sx1[cs][0]
            cy1 = sy1[cs][0]
            cx2 = sx2[cs][0]
            cy2 = sy2[cs][0]
            car = sar[cs][0]

            def ch_body(m, sup):
                ms = pl.ds(m * 16, 16)
                ix1 = jnp.maximum(kx1[ms], cx1)
                iy1 = jnp.maximum(ky1[ms], cy1)
                ix2 = jnp.minimum(kx2[ms], cx2)
                iy2 = jnp.minimum(ky2[ms], cy2)
                inter = (jnp.maximum(ix2 - ix1, 0.0)
                         * jnp.maximum(iy2 - iy1, 0.0))
                union = kar[ms] + car - inter
                iou = inter / union
                return sup | jnp.any(iou > _NMS_T)

            nchunks = (nk + 15) // 16
            sup = lax.fori_loop(0, nchunks, ch_body, False)

            @pl.when(jnp.logical_not(sup))
            def _append():
                nkv = jnp.full((16,), nk, jnp.int32)
                plsc.store_scatter(kx1, [nkv], jnp.full((16,), cx1), mask=lane0)
                plsc.store_scatter(ky1, [nkv], jnp.full((16,), cy1), mask=lane0)
                plsc.store_scatter(kx2, [nkv], jnp.full((16,), cx2), mask=lane0)
                plsc.store_scatter(ky2, [nkv], jnp.full((16,), cy2), mask=lane0)
                plsc.store_scatter(kar, [nkv], jnp.full((16,), car), mask=lane0)
                plsc.store_scatter(oscr, [nkv], jnp.full((16,), csc), mask=lane0)
                plsc.store_scatter(olab, [nkv], jnp.full((16,), clb), mask=lane0)
                bv = jnp.where(lanes == 0, cx1,
                               jnp.where(lanes == 1, cy1,
                                         jnp.where(lanes == 2, cx2, cy2)))
                plsc.store_scatter(obox, [nk * 4 + lanes], bv, mask=lanes < 4)

            nk2 = nk + jnp.where(sup, 0, 1)
            ci2 = ci + 1
            go2 = (ci2 < _N) & (nk2 < _MAXDET) & (sr[pl.ds(ci2, 16)][0] > _NEG_INF)
            return ci2, nk2, go2

        go0 = sr[pl.ds(0, 16)][0] > _NEG_INF
        lax.while_loop(nms_cond, nms_body,
                       (jnp.int32(0), jnp.int32(0), go0))

        pltpu.sync_copy(obox, ob_h.at[bb])
        pltpu.sync_copy(oscr, os_h.at[bb])
        pltpu.sync_copy(olab, ol_h.at[bb])


def _sc_nms(skey, sidx, lab, boxt):
    mesh = plsc.VectorSubcoreMesh(core_axis_name="c", subcore_axis_name="s")
    f = pl.kernel(
        _sc_body,
        out_type=[
            jax.ShapeDtypeStruct((_B, _KCAP * 4), jnp.float32),
            jax.ShapeDtypeStruct((_B, _KCAP), jnp.float32),
            jax.ShapeDtypeStruct((_B, _KCAP), jnp.int32),
        ],
        mesh=mesh,
        compiler_params=pltpu.CompilerParams(needs_layout_passes=False,
                                             use_tc_tiling_on_sc=False),
        scratch_types=[
            pltpu.VMEM((_N16,), jnp.int32),
            pltpu.VMEM((_N16,), jnp.int32),
            pltpu.VMEM((4, _N16), jnp.float32),
            pltpu.VMEM((_NS,), jnp.float32),
            pltpu.VMEM((_NS,), jnp.int32),
            pltpu.VMEM((_NS,), jnp.float32),
            pltpu.VMEM((_NS,), jnp.float32),
            pltpu.VMEM((_NS,), jnp.float32),
            pltpu.VMEM((_NS,), jnp.float32),
            pltpu.VMEM((_NS,), jnp.float32),
            pltpu.VMEM((_KCAP,), jnp.float32),
            pltpu.VMEM((_KCAP,), jnp.float32),
            pltpu.VMEM((_KCAP,), jnp.float32),
            pltpu.VMEM((_KCAP,), jnp.float32),
            pltpu.VMEM((_KCAP,), jnp.float32),
            pltpu.VMEM((_KCAP * 4,), jnp.float32),
            pltpu.VMEM((_KCAP,), jnp.float32),
            pltpu.VMEM((_KCAP,), jnp.int32),
        ],
    )
    return f(skey, sidx, lab, boxt)


def kernel(boxes, classification, centerness):
    boxt = jnp.transpose(boxes, (0, 2, 1))
    boxt = jnp.concatenate(
        [boxt, jnp.zeros((_B, 4, _N16 - _N), jnp.float32)], axis=2)
    cen = centerness[..., 0]
    skey, sidx, lab = _prep(classification, cen)
    ob, osr, olb = _sc_nms(skey, sidx, lab, boxt)
    boxes_out = ob.reshape(_B, _KCAP, 4)[:, :_MAXDET, :]
    scores_out = osr[:, :_MAXDET]
    labels_out = olb[:, :_MAXDET]
    return boxes_out, scores_out, labels_out


# speedup vs baseline: 1640.4697x; 1.6156x over previous
import functools

import jax
import jax.numpy as jnp
from jax import lax
from jax.experimental import pallas as pl
from jax.experimental.pallas import tpu as pltpu
from jax.experimental.pallas import tpu_sc as plsc

_B, _N, _C = 8, 5000, 80
_SCORE_T = 0.05
_NMS_T = 0.5
_MAXDET = 300
_NSORT = 8192
_BR = 64
_SR, _SC = 512, 128
_N16 = 5008
_NS = 5024
_KCAP = 304

_NEG_INF = float("-inf")


def _partner(x, j, axis):
    lo = jnp.roll(x, -j, axis=axis)
    hi = jnp.roll(x, j, axis=axis)
    idx = lax.broadcasted_iota(jnp.int32, (_SR, _SC), axis)
    return jnp.where((idx & j) == 0, lo, hi), (idx & j) == 0


def _reduce_body(cls_ref, cen_ref, rank_ref, lab_ref):
    cls = cls_ref[...]
    raw = jnp.max(cls, axis=0)
    iota_c = lax.broadcasted_iota(jnp.int32, (_C, _N), 0)
    lab = jnp.min(jnp.where(cls == raw[None, :], iota_c, _C), axis=0)
    cen = cen_ref[0, :]
    comb = jnp.sqrt(raw * cen)
    valid = raw > _SCORE_T
    rank_ref[0, :] = jnp.where(valid, comb, _NEG_INF)
    lab_ref[0, :] = lab


def _sort_body(rank_ref, skey_ref, sidx_ref):
    kneg = lax.bitcast_convert_type(jnp.float32(_NEG_INF), jnp.int32)
    krows = []
    for b in range(_B):
        k = lax.bitcast_convert_type(rank_ref[b], jnp.int32)
        krows.append(jnp.concatenate(
            [k, jnp.full((_NSORT - _N,), kneg, jnp.int32)]).reshape(_BR, _SC))

    K = jnp.concatenate(krows, axis=0)
    row_i = lax.broadcasted_iota(jnp.int32, (_SR, _SC), 0)
    lane_i = lax.broadcasted_iota(jnp.int32, (_SR, _SC), 1)
    brow_i = row_i & (_BR - 1)
    I = brow_i * _SC + lane_i

    kk = 2
    while kk <= _NSORT:
        if kk >= _SC:
            up = (brow_i & (kk // _SC)) == 0
        else:
            up = (lane_i & kk) == 0
        jj = kk // 2
        while jj >= 1:
            if jj >= _SC:
                pK, lower = _partner(K, jj // _SC, 0)
                pI, _ = _partner(I, jj // _SC, 0)
            else:
                pK, lower = _partner(K, jj, 1)
                pI, _ = _partner(I, jj, 1)
            a_lt_b = (K > pK) | ((K == pK) & (I < pI))
            keep_small = lower == up
            take_self = a_lt_b == keep_small
            K = jnp.where(take_self, K, pK)
            I = jnp.where(take_self, I, pI)
            jj //= 2
        kk *= 2

    skey_ref[...] = lax.bitcast_convert_type(K, jnp.float32)
    sidx_ref[...] = I


def _prep(classification, centerness):
    rank, lab = pl.pallas_call(
        _reduce_body,
        grid=(_B,),
        in_specs=[
            pl.BlockSpec((None, _C, _N), lambda b: (b, 0, 0)),
            pl.BlockSpec((None, 1, _N), lambda b: (b, 0, 0)),
        ],
        out_specs=[
            pl.BlockSpec((None, 1, _N), lambda b: (b, 0, 0)),
            pl.BlockSpec((None, 1, _N), lambda b: (b, 0, 0)),
        ],
        out_shape=[
            jax.ShapeDtypeStruct((_B, 1, _N), jnp.float32),
            jax.ShapeDtypeStruct((_B, 1, _N), jnp.int32),
        ],
    )(jnp.transpose(classification, (0, 2, 1)), centerness[:, None, :])
    rank = rank.reshape(_B, _N)
    skey, sidx = pl.pallas_call(
        _sort_body,
        out_shape=[
            jax.ShapeDtypeStruct((_SR, _SC), jnp.float32),
            jax.ShapeDtypeStruct((_SR, _SC), jnp.int32),
        ],
    )(rank)
    return (skey.reshape(_B, _NSORT), sidx.reshape(_B, _NSORT),
            lab.reshape(_B, _N))


def _sc_body(skey_h, sidx_h, lab_h, boxt_h,
             ob_h, os_h, ol_h,
             in_sidx, in_lab, in_box,
             sr, slb, sx1, sy1, sx2, sy2, sar,
             kx1, ky1, kx2, ky2, kar,
             obox, oscr, olab):
    bb = lax.axis_index("c") + 2 * lax.axis_index("s")

    @pl.when(bb < _B)
    def _work():
        pltpu.sync_copy(skey_h.at[bb, pl.ds(0, _NS)], sr)
        pltpu.sync_copy(sidx_h.at[bb, pl.ds(0, _N16)], in_sidx)
        pltpu.sync_copy(lab_h.at[bb], in_lab.at[pl.ds(0, _N)])
        pltpu.sync_copy(boxt_h.at[bb], in_box)

        lanes = lax.iota(jnp.int32, 16)
        lane0 = lanes == 0
        neg1f = jnp.full((16,), -1.0, jnp.float32)
        neg1i = jnp.full((16,), -1, jnp.int32)

        def init_k(m, carry):
            ms = pl.ds(m * 16, 16)
            kx1[ms] = neg1f
            ky1[ms] = neg1f
            kx2[ms] = neg1f
            ky2[ms] = neg1f
            kar[ms] = jnp.zeros((16,), jnp.float32)
            return carry

        lax.fori_loop(0, _KCAP // 16, init_k, 0)

        def init_o(m, carry):
            ms = pl.ds(m * 16, 16)
            oscr[ms] = neg1f
            olab[ms] = neg1i
            return carry

        lax.fori_loop(0, _KCAP // 16, init_o, 0)

        def init_ob(m, carry):
            obox[pl.ds(m * 16, 16)] = neg1f
            return carry

        lax.fori_loop(0, (_KCAP * 4) // 16, init_ob, 0)

        row0 = jnp.zeros((16,), jnp.int32)

        def gather_chunk(k, carry):
            ms = pl.ds(k * 16, 16)
            idx = in_sidx[ms]
            slb[ms] = plsc.load_gather(in_lab, [idx])
            x1 = plsc.load_gather(in_box, [row0, idx])
            y1 = plsc.load_gather(in_box, [row0 + 1, idx])
            x2 = plsc.load_gather(in_box, [row0 + 2, idx])
            y2 = plsc.load_gather(in_box, [row0 + 3, idx])
            sx1[ms] = x1
            sy1[ms] = y1
            sx2[ms] = x2
            sy2[ms] = y2
            sar[ms] = (x2 - x1) * (y2 - y1)
            return carry

        lax.fori_loop(0, _N16 // 16, gather_chunk, 0)

        def nms_cond(st):
            return st[2]

        def nms_body(st):
            ci, nk, _go = st
            cs = pl.ds(ci, 16)
            csc = sr[cs][0]
            clb = slb[cs][0]
            cx1 = sx1[cs][0]
            cy1 = sy1[cs][0]
            cx2 = sx2[cs][0]
            cy2 = sy2[cs][0]
            car = sar[cs][0]

            def ch_body(m, sup):
                ms = pl.ds(m * 16, 16)
                ix1 = jnp.maximum(kx1[ms], cx1)
                iy1 = jnp.maximum(ky1[ms], cy1)
                ix2 = jnp.minimum(kx2[ms], cx2)
                iy2 = jnp.minimum(ky2[ms], cy2)
                inter = (jnp.maximum(ix2 - ix1, 0.0)
                         * jnp.maximum(iy2 - iy1, 0.0))
                union = kar[ms] + car - inter
                iou = inter / union
                return sup | jnp.any(iou > _NMS_T)

            nchunks = (nk + 15) // 16
            sup = lax.fori_loop(0, nchunks, ch_body, False)

            @pl.when(jnp.logical_not(sup))
            def _append():
                nkv = jnp.full((16,), nk, jnp.int32)
                plsc.store_scatter(kx1, [nkv], jnp.full((16,), cx1), mask=lane0)
                plsc.store_scatter(ky1, [nkv], jnp.full((16,), cy1), mask=lane0)
                plsc.store_scatter(kx2, [nkv], jnp.full((16,), cx2), mask=lane0)
                plsc.store_scatter(ky2, [nkv], jnp.full((16,), cy2), mask=lane0)
                plsc.store_scatter(kar, [nkv], jnp.full((16,), car), mask=lane0)
                plsc.store_scatter(oscr, [nkv], jnp.full((16,), csc), mask=lane0)
                plsc.store_scatter(olab, [nkv], jnp.full((16,), clb), mask=lane0)
                bv = jnp.where(lanes == 0, cx1,
                               jnp.where(lanes == 1, cy1,
                                         jnp.where(lanes == 2, cx2, cy2)))
                plsc.store_scatter(obox, [nk * 4 + lanes], bv, mask=lanes < 4)

            nk2 = nk + jnp.where(sup, 0, 1)
            ci2 = ci + 1
            go2 = (ci2 < _N) & (nk2 < _MAXDET) & (sr[pl.ds(ci2, 16)][0] > _NEG_INF)
            return ci2, nk2, go2

        go0 = sr[pl.ds(0, 16)][0] > _NEG_INF
        lax.while_loop(nms_cond, nms_body,
                       (jnp.int32(0), jnp.int32(0), go0))

        pltpu.sync_copy(obox, ob_h.at[bb])
        pltpu.sync_copy(oscr, os_h.at[bb])
        pltpu.sync_copy(olab, ol_h.at[bb])


def _sc_nms(skey, sidx, lab, boxt):
    mesh = plsc.VectorSubcoreMesh(core_axis_name="c", subcore_axis_name="s")
    f = pl.kernel(
        _sc_body,
        out_type=[
            jax.ShapeDtypeStruct((_B, _KCAP * 4), jnp.float32),
            jax.ShapeDtypeStruct((_B, _KCAP), jnp.float32),
            jax.ShapeDtypeStruct((_B, _KCAP), jnp.int32),
        ],
        mesh=mesh,
        compiler_params=pltpu.CompilerParams(needs_layout_passes=False,
                                             use_tc_tiling_on_sc=False),
        scratch_types=[
            pltpu.VMEM((_N16,), jnp.int32),
            pltpu.VMEM((_N16,), jnp.int32),
            pltpu.VMEM((4, _N16), jnp.float32),
            pltpu.VMEM((_NS,), jnp.float32),
            pltpu.VMEM((_NS,), jnp.int32),
            pltpu.VMEM((_NS,), jnp.float32),
            pltpu.VMEM((_NS,), jnp.float32),
            pltpu.VMEM((_NS,), jnp.float32),
            pltpu.VMEM((_NS,), jnp.float32),
            pltpu.VMEM((_NS,), jnp.float32),
            pltpu.VMEM((_KCAP,), jnp.float32),
            pltpu.VMEM((_KCAP,), jnp.float32),
            pltpu.VMEM((_KCAP,), jnp.float32),
            pltpu.VMEM((_KCAP,), jnp.float32),
            pltpu.VMEM((_KCAP,), jnp.float32),
            pltpu.VMEM((_KCAP * 4,), jnp.float32),
            pltpu.VMEM((_KCAP,), jnp.float32),
            pltpu.VMEM((_KCAP,), jnp.int32),
        ],
    )
    return f(skey, sidx, lab, boxt)


def kernel(boxes, classification, centerness):
    boxt = jnp.transpose(boxes, (0, 2, 1))
    boxt = jnp.concatenate(
        [boxt, jnp.zeros((_B, 4, _N16 - _N), jnp.float32)], axis=2)
    cen = centerness[..., 0]
    skey, sidx, lab = _prep(classification, cen)
    ob, osr, olb = _sc_nms(skey, sidx, lab, boxt)
    boxes_out = ob.reshape(_B, _KCAP, 4)[:, :_MAXDET, :]
    scores_out = osr[:, :_MAXDET]
    labels_out = olb[:, :_MAXDET]
    return boxes_out, scores_out, labels_out


# speedup vs baseline: 1736.0119x; 1.0582x over previous
import functools

import jax
import jax.numpy as jnp
from jax import lax
from jax.experimental import pallas as pl
from jax.experimental.pallas import tpu as pltpu
from jax.experimental.pallas import tpu_sc as plsc

_B, _N, _C = 8, 5000, 80
_SCORE_T = 0.05
_NMS_T = 0.5
_MAXDET = 300
_NSORT = 8192
_BR = 64
_SR, _SC = 512, 128
_N16 = 5008
_NS = 5024
_KCAP = 320

_NEG_INF = float("-inf")


def _partner(x, j, axis):
    lo = jnp.roll(x, -j, axis=axis)
    hi = jnp.roll(x, j, axis=axis)
    idx = lax.broadcasted_iota(jnp.int32, (_SR, _SC), axis)
    return jnp.where((idx & j) == 0, lo, hi), (idx & j) == 0


def _reduce_body(cls_ref, cen_ref, rank_ref, lab_ref):
    cls = cls_ref[...]
    raw = jnp.max(cls, axis=0)
    iota_c = lax.broadcasted_iota(jnp.int32, (_C, _N), 0)
    lab = jnp.min(jnp.where(cls == raw[None, :], iota_c, _C), axis=0)
    cen = cen_ref[0, :]
    comb = jnp.sqrt(raw * cen)
    valid = raw > _SCORE_T
    rank_ref[0, :] = jnp.where(valid, comb, _NEG_INF)
    lab_ref[0, :] = lab


def _sort_body(rank_ref, skey_ref, sidx_ref):
    kneg = lax.bitcast_convert_type(jnp.float32(_NEG_INF), jnp.int32)
    krows = []
    for b in range(_B):
        k = lax.bitcast_convert_type(rank_ref[b], jnp.int32)
        krows.append(jnp.concatenate(
            [k, jnp.full((_NSORT - _N,), kneg, jnp.int32)]).reshape(_BR, _SC))

    K = jnp.concatenate(krows, axis=0)
    row_i = lax.broadcasted_iota(jnp.int32, (_SR, _SC), 0)
    lane_i = lax.broadcasted_iota(jnp.int32, (_SR, _SC), 1)
    brow_i = row_i & (_BR - 1)
    I = brow_i * _SC + lane_i

    kk = 2
    while kk <= _NSORT:
        if kk >= _SC:
            up = (brow_i & (kk // _SC)) == 0
        else:
            up = (lane_i & kk) == 0
        jj = kk // 2
        while jj >= 1:
            if jj >= _SC:
                pK, lower = _partner(K, jj // _SC, 0)
                pI, _ = _partner(I, jj // _SC, 0)
            else:
                pK, lower = _partner(K, jj, 1)
                pI, _ = _partner(I, jj, 1)
            a_lt_b = (K > pK) | ((K == pK) & (I < pI))
            keep_small = lower == up
            take_self = a_lt_b == keep_small
            K = jnp.where(take_self, K, pK)
            I = jnp.where(take_self, I, pI)
            jj //= 2
        kk *= 2

    skey_ref[...] = lax.bitcast_convert_type(K, jnp.float32)
    sidx_ref[...] = I


def _prep(classification, centerness):
    rank, lab = pl.pallas_call(
        _reduce_body,
        grid=(_B,),
        in_specs=[
            pl.BlockSpec((None, _C, _N), lambda b: (b, 0, 0)),
            pl.BlockSpec((None, 1, _N), lambda b: (b, 0, 0)),
        ],
        out_specs=[
            pl.BlockSpec((None, 1, _N), lambda b: (b, 0, 0)),
            pl.BlockSpec((None, 1, _N), lambda b: (b, 0, 0)),
        ],
        out_shape=[
            jax.ShapeDtypeStruct((_B, 1, _N), jnp.float32),
            jax.ShapeDtypeStruct((_B, 1, _N), jnp.int32),
        ],
    )(jnp.transpose(classification, (0, 2, 1)), centerness[:, None, :])
    rank = rank.reshape(_B, _N)
    skey, sidx = pl.pallas_call(
        _sort_body,
        out_shape=[
            jax.ShapeDtypeStruct((_SR, _SC), jnp.float32),
            jax.ShapeDtypeStruct((_SR, _SC), jnp.int32),
        ],
    )(rank)
    return (skey.reshape(_B, _NSORT), sidx.reshape(_B, _NSORT),
            lab.reshape(_B, _N))


def _sc_body(skey_h, sidx_h, lab_h, boxt_h,
             ob_h, os_h, ol_h,
             in_sidx, in_lab, in_box,
             sr, slb, sx1, sy1, sx2, sy2, sar,
             kx1, ky1, kx2, ky2, kar,
             obox, oscr, olab):
    bb = lax.axis_index("c") + 2 * lax.axis_index("s")

    @pl.when(bb < _B)
    def _work():
        pltpu.sync_copy(skey_h.at[bb, pl.ds(0, _NS)], sr)
        pltpu.sync_copy(sidx_h.at[bb, pl.ds(0, _N16)], in_sidx)
        pltpu.sync_copy(lab_h.at[bb], in_lab.at[pl.ds(0, _N)])
        pltpu.sync_copy(boxt_h.at[bb], in_box)

        lanes = lax.iota(jnp.int32, 16)
        lane0 = lanes == 0
        neg1f = jnp.full((16,), -1.0, jnp.float32)
        neg1i = jnp.full((16,), -1, jnp.int32)

        def init_k(m, carry):
            ms = pl.ds(m * 16, 16)
            kx1[ms] = neg1f
            ky1[ms] = neg1f
            kx2[ms] = neg1f
            ky2[ms] = neg1f
            kar[ms] = jnp.zeros((16,), jnp.float32)
            return carry

        lax.fori_loop(0, _KCAP // 16, init_k, 0)

        def init_o(m, carry):
            ms = pl.ds(m * 16, 16)
            oscr[ms] = neg1f
            olab[ms] = neg1i
            return carry

        lax.fori_loop(0, _KCAP // 16, init_o, 0)

        def init_ob(m, carry):
            obox[pl.ds(m * 16, 16)] = neg1f
            return carry

        lax.fori_loop(0, (_KCAP * 4) // 16, init_ob, 0)

        row0 = jnp.zeros((16,), jnp.int32)

        def gather_chunk(k, carry):
            ms = pl.ds(k * 16, 16)
            idx = in_sidx[ms]
            slb[ms] = plsc.load_gather(in_lab, [idx])
            x1 = plsc.load_gather(in_box, [row0, idx])
            y1 = plsc.load_gather(in_box, [row0 + 1, idx])
            x2 = plsc.load_gather(in_box, [row0 + 2, idx])
            y2 = plsc.load_gather(in_box, [row0 + 3, idx])
            sx1[ms] = x1
            sy1[ms] = y1
            sx2[ms] = x2
            sy2[ms] = y2
            sar[ms] = (x2 - x1) * (y2 - y1)
            return carry

        lax.fori_loop(0, _N16 // 16, gather_chunk, 0)

        def nms_cond(st):
            return st[2]

        def nms_body(st):
            ci, nk, _go = st
            cs = pl.ds(ci, 16)
            csc = sr[cs][0]
            clb = slb[cs][0]
            cx1 = sx1[cs][0]
            cy1 = sy1[cs][0]
            cx2 = sx2[cs][0]
            cy2 = sy2[cs][0]
            car = sar[cs][0]

            def ch_half(ms, supv):
                ix1 = jnp.maximum(kx1[ms], cx1)
                iy1 = jnp.maximum(ky1[ms], cy1)
                ix2 = jnp.minimum(kx2[ms], cx2)
                iy2 = jnp.minimum(ky2[ms], cy2)
                inter = (jnp.maximum(ix2 - ix1, 0.0)
                         * jnp.maximum(iy2 - iy1, 0.0))
                union = kar[ms] + car - inter
                iou = inter / union
                return supv | (iou > _NMS_T)

            def ch_body(m, supv):
                supv = ch_half(pl.ds(m * 32, 16), supv)
                return ch_half(pl.ds(m * 32 + 16, 16), supv)

            nchunks = (nk + 31) // 32
            supv = lax.fori_loop(0, nchunks, ch_body,
                                 jnp.zeros((16,), jnp.bool_))
            sup = jnp.any(supv)

            @pl.when(jnp.logical_not(sup))
            def _append():
                nkv = jnp.full((16,), nk, jnp.int32)
                plsc.store_scatter(kx1, [nkv], jnp.full((16,), cx1), mask=lane0)
                plsc.store_scatter(ky1, [nkv], jnp.full((16,), cy1), mask=lane0)
                plsc.store_scatter(kx2, [nkv], jnp.full((16,), cx2), mask=lane0)
                plsc.store_scatter(ky2, [nkv], jnp.full((16,), cy2), mask=lane0)
                plsc.store_scatter(kar, [nkv], jnp.full((16,), car), mask=lane0)
                plsc.store_scatter(oscr, [nkv], jnp.full((16,), csc), mask=lane0)
                plsc.store_scatter(olab, [nkv], jnp.full((16,), clb), mask=lane0)
                bv = jnp.where(lanes == 0, cx1,
                               jnp.where(lanes == 1, cy1,
                                         jnp.where(lanes == 2, cx2, cy2)))
                plsc.store_scatter(obox, [nk * 4 + lanes], bv, mask=lanes < 4)

            nk2 = nk + jnp.where(sup, 0, 1)
            ci2 = ci + 1
            go2 = (ci2 < _N) & (nk2 < _MAXDET) & (sr[pl.ds(ci2, 16)][0] > _NEG_INF)
            return ci2, nk2, go2

        go0 = sr[pl.ds(0, 16)][0] > _NEG_INF
        lax.while_loop(nms_cond, nms_body,
                       (jnp.int32(0), jnp.int32(0), go0))

        pltpu.sync_copy(obox, ob_h.at[bb])
        pltpu.sync_copy(oscr, os_h.at[bb])
        pltpu.sync_copy(olab, ol_h.at[bb])


def _sc_nms(skey, sidx, lab, boxt):
    mesh = plsc.VectorSubcoreMesh(core_axis_name="c", subcore_axis_name="s")
    f = pl.kernel(
        _sc_body,
        out_type=[
            jax.ShapeDtypeStruct((_B, _KCAP * 4), jnp.float32),
            jax.ShapeDtypeStruct((_B, _KCAP), jnp.float32),
            jax.ShapeDtypeStruct((_B, _KCAP), jnp.int32),
        ],
        mesh=mesh,
        compiler_params=pltpu.CompilerParams(needs_layout_passes=False,
                                             use_tc_tiling_on_sc=False),
        scratch_types=[
            pltpu.VMEM((_N16,), jnp.int32),
            pltpu.VMEM((_N16,), jnp.int32),
            pltpu.VMEM((4, _N16), jnp.float32),
            pltpu.VMEM((_NS,), jnp.float32),
            pltpu.VMEM((_NS,), jnp.int32),
            pltpu.VMEM((_NS,), jnp.float32),
            pltpu.VMEM((_NS,), jnp.float32),
            pltpu.VMEM((_NS,), jnp.float32),
            pltpu.VMEM((_NS,), jnp.float32),
            pltpu.VMEM((_NS,), jnp.float32),
            pltpu.VMEM((_KCAP,), jnp.float32),
            pltpu.VMEM((_KCAP,), jnp.float32),
            pltpu.VMEM((_KCAP,), jnp.float32),
            pltpu.VMEM((_KCAP,), jnp.float32),
            pltpu.VMEM((_KCAP,), jnp.float32),
            pltpu.VMEM((_KCAP * 4,), jnp.float32),
            pltpu.VMEM((_KCAP,), jnp.float32),
            pltpu.VMEM((_KCAP,), jnp.int32),
        ],
    )
    return f(skey, sidx, lab, boxt)


def kernel(boxes, classification, centerness):
    boxt = jnp.transpose(boxes, (0, 2, 1))
    boxt = jnp.concatenate(
        [boxt, jnp.zeros((_B, 4, _N16 - _N), jnp.float32)], axis=2)
    cen = centerness[..., 0]
    skey, sidx, lab = _prep(classification, cen)
    ob, osr, olb = _sc_nms(skey, sidx, lab, boxt)
    boxes_out = ob.reshape(_B, _KCAP, 4)[:, :_MAXDET, :]
    scores_out = osr[:, :_MAXDET]
    labels_out = olb[:, :_MAXDET]
    return boxes_out, scores_out, labels_out
